# baseline (speedup 1.0000x reference)
import jax
import jax.numpy as jnp
from jax.experimental import pallas as pl
from jax.experimental.pallas import tpu as pltpu

_HI = jax.lax.Precision.HIGHEST
_BF = jnp.bfloat16
_F8 = jnp.float8_e4m3fn


def _lrelu(v):
    return jnp.where(v >= 0, v, 0.2 * v)


def _s1_body(x_ref, w1_ref, s1b_ref):
    s1b_ref[...] = _lrelu(jnp.dot(x_ref[...], w1_ref[...], precision=_HI,
                                  preferred_element_type=jnp.float32)).astype(_BF)


def _pa_body(adj_ref, s1b_ref, w2_ref,
             z1_ref, z1b_ref, s2b_ref, adjf8_ref):
    a = adj_ref[...]
    adjf8_ref[...] = a.astype(_F8)
    z1 = jnp.dot(a.astype(_BF), s1b_ref[...],
                 preferred_element_type=jnp.float32)
    z1_ref[...] = z1
    z1b_ref[...] = z1.astype(_BF)
    s2 = _lrelu(jnp.dot(z1, w2_ref[...], precision=_HI,
                        preferred_element_type=jnp.float32))
    s2b_ref[...] = s2.astype(_BF)


def _pb_body(adjf8_ref, s2b_ref, z1b_ref, w3_ref, z2_ref, c12b_ref, s3b_ref):
    z2 = jnp.dot(adjf8_ref[...].astype(_BF), s2b_ref[...],
                 preferred_element_type=jnp.float32)
    z2_ref[...] = z2
    c12b_ref[...] = jnp.concatenate([z1b_ref[...], z2.astype(_BF)], axis=1)
    s3 = jnp.dot(z2, w3_ref[...], precision=_HI,
                 preferred_element_type=jnp.float32)
    s3b_ref[...] = s3.astype(_BF)


def _pc_body(adjf8_ref, s3b_ref, c12b_ref, zi_ref, call_ref):
    zi = jnp.dot(adjf8_ref[...].astype(_BF), s3b_ref[...],
                 preferred_element_type=jnp.float32)
    zi_ref[...] = zi
    call_ref[...] = jnp.concatenate([c12b_ref[...], zi.astype(_BF)], axis=1)


def _pd_body(h1, h2, nz, adjf8_ref, call_ref, crow_ref,
             zadj_ref, az1_ref, az2_ref, az3_ref):
    cal = call_ref[...]
    zrb = crow_ref[...][:, h1 + h2:]
    zcb = cal[:, h1 + h2:]
    g = jax.lax.dot_general(zrb, zcb, (((1,), (1,)), ((), ())),
                            preferred_element_type=jnp.float32)
    zadj_ref[...] = 0.5 * (jnp.tanh(0.5 * g) + 1.0)
    r = jnp.dot(adjf8_ref[...].astype(_BF), cal,
                preferred_element_type=jnp.float32)
    az1_ref[...] = r[:, :h1]
    az2_ref[...] = r[:, h1:h1 + h2]
    az3_ref[...] = r[:, h1 + h2:]


def kernel(x, adj, W1, W2, W3):
    n, d_in = x.shape
    h1 = W1.shape[1]
    h2 = W2.shape[1]
    nz = W3.shape[1]
    w_all = h1 + h2 + nz
    f32 = jnp.float32

    bm_s = n // 5
    s1b = pl.pallas_call(
        _s1_body,
        grid=(n // bm_s,),
        in_specs=[pl.BlockSpec((bm_s, d_in), lambda i: (i, 0)),
                  pl.BlockSpec((d_in, h1), lambda i: (0, 0))],
        out_specs=pl.BlockSpec((bm_s, h1), lambda i: (i, 0)),
        out_shape=jax.ShapeDtypeStruct((n, h1), _BF),
    )(x, W1)

    bm_a = n // 25
    z1, z1b, s2b, adjf8 = pl.pallas_call(
        _pa_body,
        grid=(n // bm_a,),
        in_specs=[pl.BlockSpec((bm_a, n), lambda i: (i, 0)),
                  pl.BlockSpec((n, h1), lambda i: (0, 0)),
                  pl.BlockSpec((h1, h2), lambda i: (0, 0))],
        out_specs=[pl.BlockSpec((bm_a, h1), lambda i: (i, 0)),
                   pl.BlockSpec((bm_a, h1), lambda i: (i, 0)),
                   pl.BlockSpec((bm_a, h2), lambda i: (i, 0)),
                   pl.BlockSpec((bm_a, n), lambda i: (i, 0))],
        out_shape=[jax.ShapeDtypeStruct((n, h1), f32),
                   jax.ShapeDtypeStruct((n, h1), _BF),
                   jax.ShapeDtypeStruct((n, h2), _BF),
                   jax.ShapeDtypeStruct((n, n), _F8)],
    )(adj, s1b, W2)

    bm_b = n // 5
    z2, c12b, s3b = pl.pallas_call(
        _pb_body,
        grid=(n // bm_b,),
        in_specs=[pl.BlockSpec((bm_b, n), lambda i: (i, 0)),
                  pl.BlockSpec((n, h2), lambda i: (0, 0)),
                  pl.BlockSpec((bm_b, h1), lambda i: (i, 0)),
                  pl.BlockSpec((h2, nz), lambda i: (0, 0))],
        out_specs=[pl.BlockSpec((bm_b, h2), lambda i: (i, 0)),
                   pl.BlockSpec((bm_b, h1 + h2), lambda i: (i, 0)),
                   pl.BlockSpec((bm_b, nz), lambda i: (i, 0))],
        out_shape=[jax.ShapeDtypeStruct((n, h2), f32),
                   jax.ShapeDtypeStruct((n, h1 + h2), _BF),
                   jax.ShapeDtypeStruct((n, nz), _BF)],
    )(adjf8, s2b, z1b, W3)

    az_w = h1 + h2 + nz
    z_igae, c_all = pl.pallas_call(
        _pc_body,
        grid=(n // bm_b,),
        in_specs=[pl.BlockSpec((bm_b, n), lambda i: (i, 0)),
                  pl.BlockSpec((n, nz), lambda i: (0, 0)),
                  pl.BlockSpec((bm_b, h1 + h2), lambda i: (i, 0))],
        out_specs=[pl.BlockSpec((bm_b, nz), lambda i: (i, 0)),
                   pl.BlockSpec((bm_b, az_w), lambda i: (i, 0))],
        out_shape=[jax.ShapeDtypeStruct((n, nz), f32),
                   jax.ShapeDtypeStruct((n, az_w), _BF)],
    )(adjf8, s3b, c12b)

    bm_d = n // 25
    z_adj, az1, az2, az3 = pl.pallas_call(
        lambda *refs: _pd_body(h1, h2, nz, *refs),
        grid=(n // bm_d,),
        in_specs=[pl.BlockSpec((bm_d, n), lambda i: (i, 0)),
                  pl.BlockSpec((n, az_w), lambda i: (0, 0)),
                  pl.BlockSpec((bm_d, az_w), lambda i: (i, 0))],
        out_specs=[pl.BlockSpec((bm_d, n), lambda i: (i, 0)),
                   pl.BlockSpec((bm_d, h1), lambda i: (i, 0)),
                   pl.BlockSpec((bm_d, h2), lambda i: (i, 0)),
                   pl.BlockSpec((bm_d, nz), lambda i: (i, 0))],
        out_shape=[jax.ShapeDtypeStruct((n, n), f32),
                   jax.ShapeDtypeStruct((n, h1), f32),
                   jax.ShapeDtypeStruct((n, h2), f32),
                   jax.ShapeDtypeStruct((n, nz), f32)],
    )(adjf8, c_all, c_all)

    return (z_igae, z_adj, az1, az2, az3, z1, z2, z_igae)
